# Initial kernel scaffold; baseline (speedup 1.0000x reference)
import jax
import jax.numpy as jnp
from jax.experimental import pallas as pl


def kernel(feat, edge_index, emb, W0, b0, beta0, W1, b1, beta1):
    raise NotImplementedError("write your pallas kernel here")


# speedup vs baseline: 3.9873x; 3.9873x over previous
import functools

import jax
import jax.numpy as jnp
from jax import lax
from jax.experimental import pallas as pl
from jax.experimental.pallas import tpu as pltpu
from jax.experimental.pallas import tpu_sc as plsc

N = 10000
E = 320000
D_FEAT = 128
D_MSG = 256
HALF = 128
EPS = 1e-3

NC = 2
NS = 16
K = 128
J = 160
T = J * K
E_PAD = NS * T
R = 10240
ROWS_PER_TILE = R // NS
TRASH = 10000


def _deg_body(idx_hbm, out_hbm, idx_v, ones_v, zero_v, acc_sh):
    c = lax.axis_index("c")
    s = lax.axis_index("s")

    def fill_z(i, _):
        zero_v[pl.ds(i * 16, 16)] = jnp.zeros((16,), jnp.float32)
        return 0

    lax.fori_loop(0, ROWS_PER_TILE // 16, fill_z, 0)

    def fill_o(i, _):
        ones_v[pl.ds(i * 16, 16)] = jnp.ones((16,), jnp.float32)
        return 0

    lax.fori_loop(0, K // 16, fill_o, 0)

    pltpu.sync_copy(idx_hbm.at[c, s], idx_v)
    pltpu.sync_copy(zero_v, acc_sh.at[pl.ds(s * ROWS_PER_TILE, ROWS_PER_TILE)])
    plsc.subcore_barrier()

    def body(j, _):
        pltpu.sync_copy(ones_v, acc_sh.at[idx_v.at[j]], add=True)
        return 0

    lax.fori_loop(0, J, body, 0)
    plsc.subcore_barrier()
    pltpu.sync_copy(acc_sh.at[pl.ds(s * ROWS_PER_TILE, ROWS_PER_TILE)],
                    out_hbm.at[c, pl.ds(s * ROWS_PER_TILE, ROWS_PER_TILE)])


@functools.lru_cache(maxsize=None)
def _get_deg_kernel():
    mesh = plsc.VectorSubcoreMesh(core_axis_name="c", subcore_axis_name="s",
                                  num_cores=NC, num_subcores=NS)
    return pl.kernel(
        _deg_body,
        out_type=jax.ShapeDtypeStruct((2, R), jnp.float32),
        mesh=mesh,
        scratch_types=[
            pltpu.VMEM((J, K), jnp.int32),
            pltpu.VMEM((K,), jnp.float32),
            pltpu.VMEM((ROWS_PER_TILE,), jnp.float32),
            pltpu.VMEM_SHARED((R,), jnp.float32),
        ],
    )


def _msg_body(src_hbm, dst_hbm, tbl_hbm, out_hbm,
              sidx_v, didx_v, rows0, rows1, acc_sh,
              semr0, semr1, sems0, sems1, semd0, semd1):
    c = lax.axis_index("c")
    s = lax.axis_index("s")

    def fill_z(i, _):
        r = i // (HALF // 16)
        cc = i % (HALF // 16)
        rows0[r, pl.ds(cc * 16, 16)] = jnp.zeros((16,), jnp.float32)
        return 0

    lax.fori_loop(0, K * HALF // 16, fill_z, 0)

    def zero_acc(i, _):
        pltpu.sync_copy(rows0, acc_sh.at[pl.ds(s * ROWS_PER_TILE + i * K, K)])
        return 0

    lax.fori_loop(0, ROWS_PER_TILE // K, zero_acc, 0)
    plsc.subcore_barrier()

    rows = (rows0, rows1)
    semr = (semr0, semr1)
    sems = (sems0, sems1)
    semd = (semd0, semd1)

    pltpu.sync_copy(src_hbm.at[c, s, 0], sidx_v.at[0])
    pltpu.sync_copy(dst_hbm.at[s, 0], didx_v.at[0])
    pltpu.async_copy(tbl_hbm.at[sidx_v.at[0]], rows0, semr0)
    pltpu.async_copy(src_hbm.at[c, s, 1], sidx_v.at[1], sems1)
    pltpu.async_copy(dst_hbm.at[s, 1], didx_v.at[1], semd1)

    def body(g, _):
        for b in range(2):
            j = 2 * g + b
            nb = 1 - b

            @pl.when(j + 1 < J)
            def _():
                pltpu.make_async_copy(src_hbm.at[c, s, j + 1], sidx_v.at[nb],
                                      sems[nb]).wait()
                pltpu.make_async_copy(dst_hbm.at[s, j + 1], didx_v.at[nb],
                                      semd[nb]).wait()
                pltpu.async_copy(tbl_hbm.at[sidx_v.at[nb]], rows[nb], semr[nb])

            pltpu.make_async_copy(tbl_hbm.at[sidx_v.at[b]], rows[b],
                                  semr[b]).wait()
            pltpu.sync_copy(rows[b], acc_sh.at[didx_v.at[b]], add=True)

            @pl.when(j + 2 < J)
            def _():
                pltpu.async_copy(src_hbm.at[c, s, j + 2], sidx_v.at[b], sems[b])
                pltpu.async_copy(dst_hbm.at[s, j + 2], didx_v.at[b], semd[b])
        return 0

    lax.fori_loop(0, J // 2, body, 0)
    plsc.subcore_barrier()
    pltpu.sync_copy(acc_sh.at[pl.ds(s * ROWS_PER_TILE, ROWS_PER_TILE)],
                    out_hbm.at[c, pl.ds(s * ROWS_PER_TILE, ROWS_PER_TILE)])


@functools.lru_cache(maxsize=None)
def _get_msg_kernel():
    mesh = plsc.VectorSubcoreMesh(core_axis_name="c", subcore_axis_name="s",
                                  num_cores=NC, num_subcores=NS)
    return pl.kernel(
        _msg_body,
        out_type=jax.ShapeDtypeStruct((2, R, HALF), jnp.float32),
        mesh=mesh,
        scratch_types=[
            pltpu.VMEM((2, K), jnp.int32),
            pltpu.VMEM((2, K), jnp.int32),
            pltpu.VMEM((K, HALF), jnp.float32),
            pltpu.VMEM((K, HALF), jnp.float32),
            pltpu.VMEM_SHARED((R, HALF), jnp.float32),
            pltpu.SemaphoreType.DMA,
            pltpu.SemaphoreType.DMA,
            pltpu.SemaphoreType.DMA,
            pltpu.SemaphoreType.DMA,
            pltpu.SemaphoreType.DMA,
            pltpu.SemaphoreType.DMA,
        ],
    )


NB = 1000
GRID = N // NB


def _tca_body(feat_ref, emb_ref, deg_ref, h0_ref, hs_ref, norms_ref):
    nrm = lax.rsqrt(jnp.maximum(deg_ref[0], 1.0))
    norms_ref[0] = nrm
    no = nrm[0]
    h0 = jnp.concatenate(
        [feat_ref[...], jnp.broadcast_to(emb_ref[...], (NB, D_MSG - D_FEAT))],
        axis=1)
    h0_ref[...] = h0
    hs = h0 * no[:, None]
    hs_ref[0] = hs[:, :HALF]
    hs_ref[1] = hs[:, HALF:]


_tca = pl.pallas_call(
    _tca_body,
    grid=(GRID,),
    in_specs=[
        pl.BlockSpec((NB, D_FEAT), lambda i: (i, 0)),
        pl.BlockSpec((1, D_MSG - D_FEAT), lambda i: (0, 0)),
        pl.BlockSpec((1, 2, NB), lambda i: (i, 0, 0)),
    ],
    out_specs=[
        pl.BlockSpec((NB, D_MSG), lambda i: (i, 0)),
        pl.BlockSpec((2, NB, HALF), lambda i: (0, i, 0)),
        pl.BlockSpec((1, 2, NB), lambda i: (i, 0, 0)),
    ],
    out_shape=[
        jax.ShapeDtypeStruct((N, D_MSG), jnp.float32),
        jax.ShapeDtypeStruct((2, N, HALF), jnp.float32),
        jax.ShapeDtypeStruct((GRID, 2, NB), jnp.float32),
    ],
)


def _tcb_body(h_ref, m0_ref, m1_ref, norms_ref, w_ref, b_ref, beta_ref,
              hn_ref, hs_ref=None):
    nin = norms_ref[0, 1]
    mb = jnp.concatenate([m0_ref[0], m1_ref[0]], axis=1) * nin[:, None]
    out = jnp.dot(mb, w_ref[...], preferred_element_type=jnp.float32)
    out = jnp.maximum(out + b_ref[...], 0.0)
    out = h_ref[...] + out
    mu = jnp.mean(out, axis=1, keepdims=True)
    var = jnp.mean((out - mu) ** 2, axis=1, keepdims=True)
    out = (out - mu) * lax.rsqrt(var + EPS) + beta_ref[...]
    hn_ref[...] = out
    if hs_ref is not None:
        no = norms_ref[0, 0]
        hs = out * no[:, None]
        hs_ref[0] = hs[:, :HALF]
        hs_ref[1] = hs[:, HALF:]


def _make_tcb(emit_hs):
    body = _tcb_body if emit_hs else functools.partial(_tcb_body, hs_ref=None)
    out_specs = [pl.BlockSpec((NB, D_MSG), lambda i: (i, 0))]
    out_shape = [jax.ShapeDtypeStruct((N, D_MSG), jnp.float32)]
    if emit_hs:
        out_specs.append(pl.BlockSpec((2, NB, HALF), lambda i: (0, i, 0)))
        out_shape.append(jax.ShapeDtypeStruct((2, N, HALF), jnp.float32))
    return pl.pallas_call(
        body,
        grid=(GRID,),
        in_specs=[
            pl.BlockSpec((NB, D_MSG), lambda i: (i, 0)),
            pl.BlockSpec((1, NB, HALF), lambda i: (0, i, 0)),
            pl.BlockSpec((1, NB, HALF), lambda i: (1, i, 0)),
            pl.BlockSpec((1, 2, NB), lambda i: (i, 0, 0)),
            pl.BlockSpec((D_MSG, D_MSG), lambda i: (0, 0)),
            pl.BlockSpec((1, D_MSG), lambda i: (0, 0)),
            pl.BlockSpec((1, D_MSG), lambda i: (0, 0)),
        ],
        out_specs=out_specs,
        out_shape=out_shape,
    )


_tcb_mid = _make_tcb(True)
_tcb_last = _make_tcb(False)


def kernel(feat, edge_index, emb, W0, b0, beta0, W1, b1, beta1):
    src = edge_index[0].astype(jnp.int32)
    dst = edge_index[1].astype(jnp.int32)
    npad = E_PAD - E

    src0 = jnp.concatenate([src, jnp.zeros((npad,), jnp.int32)])
    msg_src = jnp.stack([src0, src0 + N]).reshape(2, NS, J, K)
    dstp = jnp.concatenate([dst, jnp.full((npad,), TRASH, jnp.int32)])
    msg_dst = dstp.reshape(NS, J, K)
    deg_idx = jnp.stack([
        jnp.concatenate([src, jnp.full((npad,), TRASH, jnp.int32)]),
        dstp,
    ]).reshape(2, NS, J, K)

    deg = _get_deg_kernel()(deg_idx)
    deg3 = deg[:, :N].reshape(2, GRID, NB).transpose(1, 0, 2)
    h0, hs0, norms = _tca(feat, emb, deg3)
    _msg = _get_msg_kernel()
    mA = _msg(msg_src, msg_dst, hs0.reshape(2 * N, HALF))
    h1, hs1 = _tcb_mid(h0, mA, mA, norms, W0, b0.reshape(1, D_MSG),
                       beta0.reshape(1, D_MSG))
    mB = _msg(msg_src, msg_dst, hs1.reshape(2 * N, HALF))
    (h2,) = _tcb_last(h1, mB, mB, norms, W1, b1.reshape(1, D_MSG),
                      beta1.reshape(1, D_MSG))
    return h2


# speedup vs baseline: 4.0944x; 1.0269x over previous
import functools

import jax
import jax.numpy as jnp
from jax import lax
from jax.experimental import pallas as pl
from jax.experimental.pallas import tpu as pltpu
from jax.experimental.pallas import tpu_sc as plsc

N = 10000
E = 320000
D_FEAT = 128
D_MSG = 256
HALF = 128
EPS = 1e-3

NC = 2
NS = 16
K = 128
J = 160
T = J * K
E_PAD = NS * T
R = 10240
ROWS_PER_TILE = R // NS
TRASH = 10000


def _deg_body(idx_hbm, out_hbm, idx_v, ones_v, zero_v, acc_sh):
    c = lax.axis_index("c")
    s = lax.axis_index("s")

    def fill_z(i, _):
        zero_v[pl.ds(i * 16, 16)] = jnp.zeros((16,), jnp.float32)
        return 0

    lax.fori_loop(0, ROWS_PER_TILE // 16, fill_z, 0)

    def fill_o(i, _):
        ones_v[pl.ds(i * 16, 16)] = jnp.ones((16,), jnp.float32)
        return 0

    lax.fori_loop(0, K // 16, fill_o, 0)

    pltpu.sync_copy(idx_hbm.at[c, s], idx_v)
    pltpu.sync_copy(zero_v, acc_sh.at[pl.ds(s * ROWS_PER_TILE, ROWS_PER_TILE)])
    plsc.subcore_barrier()

    def body(j, _):
        pltpu.sync_copy(ones_v, acc_sh.at[idx_v.at[j]], add=True)
        return 0

    lax.fori_loop(0, J, body, 0)
    plsc.subcore_barrier()
    pltpu.sync_copy(acc_sh.at[pl.ds(s * ROWS_PER_TILE, ROWS_PER_TILE)],
                    out_hbm.at[c, pl.ds(s * ROWS_PER_TILE, ROWS_PER_TILE)])


@functools.lru_cache(maxsize=None)
def _get_deg_kernel():
    mesh = plsc.VectorSubcoreMesh(core_axis_name="c", subcore_axis_name="s",
                                  num_cores=NC, num_subcores=NS)
    return pl.kernel(
        _deg_body,
        out_type=jax.ShapeDtypeStruct((2, R), jnp.float32),
        mesh=mesh,
        scratch_types=[
            pltpu.VMEM((J, K), jnp.int32),
            pltpu.VMEM((K,), jnp.float32),
            pltpu.VMEM((ROWS_PER_TILE,), jnp.float32),
            pltpu.VMEM_SHARED((R,), jnp.float32),
        ],
    )


def _msg_body(src_hbm, dst_hbm, tbl_hbm, out_hbm,
              sidx_v, didx_v, rows0, rows1, acc_sh,
              semr0, semr1, semc0, semc1, sems0, sems1,
              semd0, semd1, semd2, semd3):
    c = lax.axis_index("c")
    s = lax.axis_index("s")

    def fill_z(i, _):
        r = i // (HALF // 16)
        cc = i % (HALF // 16)
        rows0[r, pl.ds(cc * 16, 16)] = jnp.zeros((16,), jnp.float32)
        return 0

    lax.fori_loop(0, K * HALF // 16, fill_z, 0)

    def zero_acc(i, _):
        pltpu.sync_copy(rows0, acc_sh.at[pl.ds(s * ROWS_PER_TILE + i * K, K)])
        return 0

    lax.fori_loop(0, ROWS_PER_TILE // K, zero_acc, 0)
    plsc.subcore_barrier()

    rows = (rows0, rows1)
    semr = (semr0, semr1)
    semc = (semc0, semc1)
    sems = (sems0, sems1)
    semd = (semd0, semd1, semd2, semd3)

    pltpu.sync_copy(src_hbm.at[c, s, 0], sidx_v.at[0])
    pltpu.sync_copy(dst_hbm.at[s, 0], didx_v.at[0])
    pltpu.async_copy(tbl_hbm.at[sidx_v.at[0]], rows0, semr0)
    pltpu.async_copy(src_hbm.at[c, s, 1], sidx_v.at[1], sems1)
    pltpu.async_copy(dst_hbm.at[s, 1], didx_v.at[1], semd1)

    def body(g, _):
        for q in range(4):
            j = 4 * g + q
            br = q % 2
            bn = (q + 1) % 2
            bd = q

            @pl.when(j + 1 < J)
            def _():
                pltpu.make_async_copy(src_hbm.at[c, s, j + 1], sidx_v.at[bn],
                                      sems[bn]).wait()
                pltpu.make_async_copy(dst_hbm.at[s, j + 1],
                                      didx_v.at[(q + 1) % 4],
                                      semd[(q + 1) % 4]).wait()
                if q == 0:
                    @pl.when(j >= 1)
                    def _():
                        pltpu.make_async_copy(
                            rows[bn], acc_sh.at[didx_v.at[(q - 1) % 4]],
                            semc[bn]).wait()
                else:
                    pltpu.make_async_copy(
                        rows[bn], acc_sh.at[didx_v.at[(q - 1) % 4]],
                        semc[bn]).wait()
                pltpu.async_copy(tbl_hbm.at[sidx_v.at[bn]], rows[bn], semr[bn])

            pltpu.make_async_copy(tbl_hbm.at[sidx_v.at[br]], rows[br],
                                  semr[br]).wait()
            pltpu.async_copy(rows[br], acc_sh.at[didx_v.at[bd]], semc[br],
                             add=True)

            @pl.when(j + 2 < J)
            def _():
                pltpu.async_copy(src_hbm.at[c, s, j + 2], sidx_v.at[br],
                                 sems[br])
                pltpu.async_copy(dst_hbm.at[s, j + 2], didx_v.at[(q + 2) % 4],
                                 semd[(q + 2) % 4])
        return 0

    lax.fori_loop(0, J // 4, body, 0)
    pltpu.make_async_copy(rows[0], acc_sh.at[didx_v.at[2]], semc[0]).wait()
    pltpu.make_async_copy(rows[1], acc_sh.at[didx_v.at[3]], semc[1]).wait()
    plsc.subcore_barrier()
    pltpu.sync_copy(acc_sh.at[pl.ds(s * ROWS_PER_TILE, ROWS_PER_TILE)],
                    out_hbm.at[c, pl.ds(s * ROWS_PER_TILE, ROWS_PER_TILE)])


@functools.lru_cache(maxsize=None)
def _get_msg_kernel():
    mesh = plsc.VectorSubcoreMesh(core_axis_name="c", subcore_axis_name="s",
                                  num_cores=NC, num_subcores=NS)
    return pl.kernel(
        _msg_body,
        out_type=jax.ShapeDtypeStruct((2, R, HALF), jnp.float32),
        mesh=mesh,
        scratch_types=[
            pltpu.VMEM((2, K), jnp.int32),
            pltpu.VMEM((4, K), jnp.int32),
            pltpu.VMEM((K, HALF), jnp.float32),
            pltpu.VMEM((K, HALF), jnp.float32),
            pltpu.VMEM_SHARED((R, HALF), jnp.float32),
        ] + [pltpu.SemaphoreType.DMA] * 10,
    )


NB = 1000
GRID = N // NB


def _tca_body(feat_ref, emb_ref, deg_ref, h0_ref, hs_ref, norms_ref):
    nrm = lax.rsqrt(jnp.maximum(deg_ref[0], 1.0))
    norms_ref[0] = nrm
    no = nrm[0]
    h0 = jnp.concatenate(
        [feat_ref[...], jnp.broadcast_to(emb_ref[...], (NB, D_MSG - D_FEAT))],
        axis=1)
    h0_ref[...] = h0
    hs = h0 * no[:, None]
    hs_ref[0] = hs[:, :HALF]
    hs_ref[1] = hs[:, HALF:]


_tca = pl.pallas_call(
    _tca_body,
    grid=(GRID,),
    in_specs=[
        pl.BlockSpec((NB, D_FEAT), lambda i: (i, 0)),
        pl.BlockSpec((1, D_MSG - D_FEAT), lambda i: (0, 0)),
        pl.BlockSpec((1, 2, NB), lambda i: (i, 0, 0)),
    ],
    out_specs=[
        pl.BlockSpec((NB, D_MSG), lambda i: (i, 0)),
        pl.BlockSpec((2, NB, HALF), lambda i: (0, i, 0)),
        pl.BlockSpec((1, 2, NB), lambda i: (i, 0, 0)),
    ],
    out_shape=[
        jax.ShapeDtypeStruct((N, D_MSG), jnp.float32),
        jax.ShapeDtypeStruct((2, N, HALF), jnp.float32),
        jax.ShapeDtypeStruct((GRID, 2, NB), jnp.float32),
    ],
)


def _tcb_body(h_ref, m0_ref, m1_ref, norms_ref, w_ref, b_ref, beta_ref,
              hn_ref, hs_ref=None):
    nin = norms_ref[0, 1]
    mb = jnp.concatenate([m0_ref[0], m1_ref[0]], axis=1) * nin[:, None]
    out = jnp.dot(mb, w_ref[...], preferred_element_type=jnp.float32)
    out = jnp.maximum(out + b_ref[...], 0.0)
    out = h_ref[...] + out
    mu = jnp.mean(out, axis=1, keepdims=True)
    var = jnp.mean((out - mu) ** 2, axis=1, keepdims=True)
    out = (out - mu) * lax.rsqrt(var + EPS) + beta_ref[...]
    hn_ref[...] = out
    if hs_ref is not None:
        no = norms_ref[0, 0]
        hs = out * no[:, None]
        hs_ref[0] = hs[:, :HALF]
        hs_ref[1] = hs[:, HALF:]


def _make_tcb(emit_hs):
    body = _tcb_body if emit_hs else functools.partial(_tcb_body, hs_ref=None)
    out_specs = [pl.BlockSpec((NB, D_MSG), lambda i: (i, 0))]
    out_shape = [jax.ShapeDtypeStruct((N, D_MSG), jnp.float32)]
    if emit_hs:
        out_specs.append(pl.BlockSpec((2, NB, HALF), lambda i: (0, i, 0)))
        out_shape.append(jax.ShapeDtypeStruct((2, N, HALF), jnp.float32))
    return pl.pallas_call(
        body,
        grid=(GRID,),
        in_specs=[
            pl.BlockSpec((NB, D_MSG), lambda i: (i, 0)),
            pl.BlockSpec((1, NB, HALF), lambda i: (0, i, 0)),
            pl.BlockSpec((1, NB, HALF), lambda i: (1, i, 0)),
            pl.BlockSpec((1, 2, NB), lambda i: (i, 0, 0)),
            pl.BlockSpec((D_MSG, D_MSG), lambda i: (0, 0)),
            pl.BlockSpec((1, D_MSG), lambda i: (0, 0)),
            pl.BlockSpec((1, D_MSG), lambda i: (0, 0)),
        ],
        out_specs=out_specs,
        out_shape=out_shape,
    )


_tcb_mid = _make_tcb(True)
_tcb_last = _make_tcb(False)


def kernel(feat, edge_index, emb, W0, b0, beta0, W1, b1, beta1):
    src = edge_index[0].astype(jnp.int32)
    dst = edge_index[1].astype(jnp.int32)
    npad = E_PAD - E

    src0 = jnp.concatenate([src, jnp.zeros((npad,), jnp.int32)])
    msg_src = jnp.stack([src0, src0 + N]).reshape(2, NS, J, K)
    dstp = jnp.concatenate([dst, jnp.full((npad,), TRASH, jnp.int32)])
    msg_dst = dstp.reshape(NS, J, K)
    deg_idx = jnp.stack([
        jnp.concatenate([src, jnp.full((npad,), TRASH, jnp.int32)]),
        dstp,
    ]).reshape(2, NS, J, K)

    deg = _get_deg_kernel()(deg_idx)
    deg3 = deg[:, :N].reshape(2, GRID, NB).transpose(1, 0, 2)
    h0, hs0, norms = _tca(feat, emb, deg3)
    _msg = _get_msg_kernel()
    mA = _msg(msg_src, msg_dst, hs0.reshape(2 * N, HALF))
    h1, hs1 = _tcb_mid(h0, mA, mA, norms, W0, b0.reshape(1, D_MSG),
                       beta0.reshape(1, D_MSG))
    mB = _msg(msg_src, msg_dst, hs1.reshape(2 * N, HALF))
    (h2,) = _tcb_last(h1, mB, mB, norms, W1, b1.reshape(1, D_MSG),
                      beta1.reshape(1, D_MSG))
    return h2


# speedup vs baseline: 4.0953x; 1.0002x over previous
import functools

import jax
import jax.numpy as jnp
from jax import lax
from jax.experimental import pallas as pl
from jax.experimental.pallas import tpu as pltpu
from jax.experimental.pallas import tpu_sc as plsc

N = 10000
E = 320000
D_FEAT = 128
D_MSG = 256
HALF = 128
EPS = 1e-3

NC = 2
NS = 16
K = 128
J = 160
T = J * K
E_PAD = NS * T
R = 10240
ROWS_PER_TILE = R // NS
TRASH = 10000


def _deg_body(idx_hbm, out_hbm, idx_v, ones_v, zero_v, acc_sh):
    c = lax.axis_index("c")
    s = lax.axis_index("s")

    def fill_z(i, _):
        zero_v[pl.ds(i * 16, 16)] = jnp.zeros((16,), jnp.float32)
        return 0

    lax.fori_loop(0, ROWS_PER_TILE // 16, fill_z, 0)

    def fill_o(i, _):
        ones_v[pl.ds(i * 16, 16)] = jnp.ones((16,), jnp.float32)
        return 0

    lax.fori_loop(0, K // 16, fill_o, 0)

    pltpu.sync_copy(idx_hbm.at[c, s], idx_v)
    pltpu.sync_copy(zero_v, acc_sh.at[pl.ds(s * ROWS_PER_TILE, ROWS_PER_TILE)])
    plsc.subcore_barrier()

    def body(j, _):
        pltpu.sync_copy(ones_v, acc_sh.at[idx_v.at[j]], add=True)
        return 0

    lax.fori_loop(0, J, body, 0)
    plsc.subcore_barrier()
    pltpu.sync_copy(acc_sh.at[pl.ds(s * ROWS_PER_TILE, ROWS_PER_TILE)],
                    out_hbm.at[c, pl.ds(s * ROWS_PER_TILE, ROWS_PER_TILE)])


@functools.lru_cache(maxsize=None)
def _get_deg_kernel():
    mesh = plsc.VectorSubcoreMesh(core_axis_name="c", subcore_axis_name="s",
                                  num_cores=NC, num_subcores=NS)
    return pl.kernel(
        _deg_body,
        out_type=jax.ShapeDtypeStruct((2, R), jnp.float32),
        mesh=mesh,
        scratch_types=[
            pltpu.VMEM((J, K), jnp.int32),
            pltpu.VMEM((K,), jnp.float32),
            pltpu.VMEM((ROWS_PER_TILE,), jnp.float32),
            pltpu.VMEM_SHARED((R,), jnp.float32),
        ],
    )


def _msg_body(src_hbm, dst_hbm, tbl_hbm, out_hbm,
              sidx_v, didx_v, rows0, rows1, acc_sh,
              semr0, semr1, semc0, semc1, sems0, sems1,
              semd0, semd1, semd2, semd3):
    c = lax.axis_index("c")
    s = lax.axis_index("s")

    def fill_z(i, _):
        r = i // (HALF // 16)
        cc = i % (HALF // 16)
        rows0[r, pl.ds(cc * 16, 16)] = jnp.zeros((16,), jnp.float32)
        return 0

    lax.fori_loop(0, K * HALF // 16, fill_z, 0)

    def zero_acc(i, _):
        pltpu.sync_copy(rows0, acc_sh.at[pl.ds(s * ROWS_PER_TILE + i * K, K)])
        return 0

    lax.fori_loop(0, ROWS_PER_TILE // K, zero_acc, 0)
    plsc.subcore_barrier()

    rows = (rows0, rows1)
    semr = (semr0, semr1)
    semc = (semc0, semc1)
    sems = (sems0, sems1)
    semd = (semd0, semd1, semd2, semd3)

    pltpu.sync_copy(src_hbm.at[c, s, 0], sidx_v.at[0])
    pltpu.sync_copy(dst_hbm.at[s, 0], didx_v.at[0])
    pltpu.async_copy(tbl_hbm.at[sidx_v.at[0, pl.ds(0, K // 2)]],
                     rows0.at[pl.ds(0, K // 2)], semr0)
    pltpu.async_copy(tbl_hbm.at[sidx_v.at[0, pl.ds(K // 2, K // 2)]],
                     rows0.at[pl.ds(K // 2, K // 2)], semr0)
    pltpu.async_copy(src_hbm.at[c, s, 1], sidx_v.at[1], sems1)
    pltpu.async_copy(dst_hbm.at[s, 1], didx_v.at[1], semd1)

    def body(g, _):
        for q in range(4):
            j = 4 * g + q
            br = q % 2
            bn = (q + 1) % 2
            bd = q

            @pl.when(j + 1 < J)
            def _():
                pltpu.make_async_copy(src_hbm.at[c, s, j + 1], sidx_v.at[bn],
                                      sems[bn]).wait()
                pltpu.make_async_copy(dst_hbm.at[s, j + 1],
                                      didx_v.at[(q + 1) % 4],
                                      semd[(q + 1) % 4]).wait()
                if q == 0:
                    @pl.when(j >= 1)
                    def _():
                        pltpu.make_async_copy(
                            rows[bn], acc_sh.at[didx_v.at[(q - 1) % 4]],
                            semc[bn]).wait()
                else:
                    pltpu.make_async_copy(
                        rows[bn], acc_sh.at[didx_v.at[(q - 1) % 4]],
                        semc[bn]).wait()
                pltpu.async_copy(tbl_hbm.at[sidx_v.at[bn, pl.ds(0, K // 2)]],
                                 rows[bn].at[pl.ds(0, K // 2)], semr[bn])
                pltpu.async_copy(tbl_hbm.at[sidx_v.at[bn, pl.ds(K // 2, K // 2)]],
                                 rows[bn].at[pl.ds(K // 2, K // 2)], semr[bn])

            pltpu.make_async_copy(tbl_hbm.at[sidx_v.at[br]], rows[br],
                                  semr[br]).wait()
            pltpu.async_copy(rows[br], acc_sh.at[didx_v.at[bd]], semc[br],
                             add=True)

            @pl.when(j + 2 < J)
            def _():
                pltpu.async_copy(src_hbm.at[c, s, j + 2], sidx_v.at[br],
                                 sems[br])
                pltpu.async_copy(dst_hbm.at[s, j + 2], didx_v.at[(q + 2) % 4],
                                 semd[(q + 2) % 4])
        return 0

    lax.fori_loop(0, J // 4, body, 0)
    pltpu.make_async_copy(rows[0], acc_sh.at[didx_v.at[2]], semc[0]).wait()
    pltpu.make_async_copy(rows[1], acc_sh.at[didx_v.at[3]], semc[1]).wait()
    plsc.subcore_barrier()
    pltpu.sync_copy(acc_sh.at[pl.ds(s * ROWS_PER_TILE, ROWS_PER_TILE)],
                    out_hbm.at[c, pl.ds(s * ROWS_PER_TILE, ROWS_PER_TILE)])


@functools.lru_cache(maxsize=None)
def _get_msg_kernel():
    mesh = plsc.VectorSubcoreMesh(core_axis_name="c", subcore_axis_name="s",
                                  num_cores=NC, num_subcores=NS)
    return pl.kernel(
        _msg_body,
        out_type=jax.ShapeDtypeStruct((2, R, HALF), jnp.float32),
        mesh=mesh,
        scratch_types=[
            pltpu.VMEM((2, K), jnp.int32),
            pltpu.VMEM((4, K), jnp.int32),
            pltpu.VMEM((K, HALF), jnp.float32),
            pltpu.VMEM((K, HALF), jnp.float32),
            pltpu.VMEM_SHARED((R, HALF), jnp.float32),
        ] + [pltpu.SemaphoreType.DMA] * 10,
    )


NB = 1000
GRID = N // NB


def _tca_body(feat_ref, emb_ref, deg_ref, h0_ref, hs_ref, norms_ref):
    nrm = lax.rsqrt(jnp.maximum(deg_ref[0], 1.0))
    norms_ref[0] = nrm
    no = nrm[0]
    h0 = jnp.concatenate(
        [feat_ref[...], jnp.broadcast_to(emb_ref[...], (NB, D_MSG - D_FEAT))],
        axis=1)
    h0_ref[...] = h0
    hs = h0 * no[:, None]
    hs_ref[0] = hs[:, :HALF]
    hs_ref[1] = hs[:, HALF:]


_tca = pl.pallas_call(
    _tca_body,
    grid=(GRID,),
    in_specs=[
        pl.BlockSpec((NB, D_FEAT), lambda i: (i, 0)),
        pl.BlockSpec((1, D_MSG - D_FEAT), lambda i: (0, 0)),
        pl.BlockSpec((1, 2, NB), lambda i: (i, 0, 0)),
    ],
    out_specs=[
        pl.BlockSpec((NB, D_MSG), lambda i: (i, 0)),
        pl.BlockSpec((2, NB, HALF), lambda i: (0, i, 0)),
        pl.BlockSpec((1, 2, NB), lambda i: (i, 0, 0)),
    ],
    out_shape=[
        jax.ShapeDtypeStruct((N, D_MSG), jnp.float32),
        jax.ShapeDtypeStruct((2, N, HALF), jnp.float32),
        jax.ShapeDtypeStruct((GRID, 2, NB), jnp.float32),
    ],
)


def _tcb_body(h_ref, m0_ref, m1_ref, norms_ref, w_ref, b_ref, beta_ref,
              hn_ref, hs_ref=None):
    nin = norms_ref[0, 1]
    mb = jnp.concatenate([m0_ref[0], m1_ref[0]], axis=1) * nin[:, None]
    out = jnp.dot(mb, w_ref[...], preferred_element_type=jnp.float32)
    out = jnp.maximum(out + b_ref[...], 0.0)
    out = h_ref[...] + out
    mu = jnp.mean(out, axis=1, keepdims=True)
    var = jnp.mean((out - mu) ** 2, axis=1, keepdims=True)
    out = (out - mu) * lax.rsqrt(var + EPS) + beta_ref[...]
    hn_ref[...] = out
    if hs_ref is not None:
        no = norms_ref[0, 0]
        hs = out * no[:, None]
        hs_ref[0] = hs[:, :HALF]
        hs_ref[1] = hs[:, HALF:]


def _make_tcb(emit_hs):
    body = _tcb_body if emit_hs else functools.partial(_tcb_body, hs_ref=None)
    out_specs = [pl.BlockSpec((NB, D_MSG), lambda i: (i, 0))]
    out_shape = [jax.ShapeDtypeStruct((N, D_MSG), jnp.float32)]
    if emit_hs:
        out_specs.append(pl.BlockSpec((2, NB, HALF), lambda i: (0, i, 0)))
        out_shape.append(jax.ShapeDtypeStruct((2, N, HALF), jnp.float32))
    return pl.pallas_call(
        body,
        grid=(GRID,),
        in_specs=[
            pl.BlockSpec((NB, D_MSG), lambda i: (i, 0)),
            pl.BlockSpec((1, NB, HALF), lambda i: (0, i, 0)),
            pl.BlockSpec((1, NB, HALF), lambda i: (1, i, 0)),
            pl.BlockSpec((1, 2, NB), lambda i: (i, 0, 0)),
            pl.BlockSpec((D_MSG, D_MSG), lambda i: (0, 0)),
            pl.BlockSpec((1, D_MSG), lambda i: (0, 0)),
            pl.BlockSpec((1, D_MSG), lambda i: (0, 0)),
        ],
        out_specs=out_specs,
        out_shape=out_shape,
    )


_tcb_mid = _make_tcb(True)
_tcb_last = _make_tcb(False)


def kernel(feat, edge_index, emb, W0, b0, beta0, W1, b1, beta1):
    src = edge_index[0].astype(jnp.int32)
    dst = edge_index[1].astype(jnp.int32)
    npad = E_PAD - E

    src0 = jnp.concatenate([src, jnp.zeros((npad,), jnp.int32)])
    msg_src = jnp.stack([src0, src0 + N]).reshape(2, NS, J, K)
    dstp = jnp.concatenate([dst, jnp.full((npad,), TRASH, jnp.int32)])
    msg_dst = dstp.reshape(NS, J, K)
    deg_idx = jnp.stack([
        jnp.concatenate([src, jnp.full((npad,), TRASH, jnp.int32)]),
        dstp,
    ]).reshape(2, NS, J, K)

    deg = _get_deg_kernel()(deg_idx)
    deg3 = deg[:, :N].reshape(2, GRID, NB).transpose(1, 0, 2)
    h0, hs0, norms = _tca(feat, emb, deg3)
    _msg = _get_msg_kernel()
    mA = _msg(msg_src, msg_dst, hs0.reshape(2 * N, HALF))
    h1, hs1 = _tcb_mid(h0, mA, mA, norms, W0, b0.reshape(1, D_MSG),
                       beta0.reshape(1, D_MSG))
    mB = _msg(msg_src, msg_dst, hs1.reshape(2 * N, HALF))
    (h2,) = _tcb_last(h1, mB, mB, norms, W1, b1.reshape(1, D_MSG),
                      beta1.reshape(1, D_MSG))
    return h2


# speedup vs baseline: 4.0971x; 1.0004x over previous
import functools

import jax
import jax.numpy as jnp
from jax import lax
from jax.experimental import pallas as pl
from jax.experimental.pallas import tpu as pltpu
from jax.experimental.pallas import tpu_sc as plsc

N = 10000
E = 320000
D_FEAT = 128
D_MSG = 256
HALF = 128
EPS = 1e-3

NC = 2
NS = 16
K = 128
J = 160
T = J * K
E_PAD = NS * T
R = 10240
ROWS_PER_TILE = R // NS
TRASH = 10000


def _deg_body(idx_hbm, out_hbm, idx_v, ones_v, zero_v, acc_sh):
    c = lax.axis_index("c")
    s = lax.axis_index("s")

    def fill_z(i, _):
        zero_v[pl.ds(i * 16, 16)] = jnp.zeros((16,), jnp.float32)
        return 0

    lax.fori_loop(0, ROWS_PER_TILE // 16, fill_z, 0)

    def fill_o(i, _):
        ones_v[pl.ds(i * 16, 16)] = jnp.ones((16,), jnp.float32)
        return 0

    lax.fori_loop(0, K // 16, fill_o, 0)

    pltpu.sync_copy(idx_hbm.at[c, s], idx_v)
    pltpu.sync_copy(zero_v, acc_sh.at[pl.ds(s * ROWS_PER_TILE, ROWS_PER_TILE)])
    plsc.subcore_barrier()

    def body(j, _):
        pltpu.sync_copy(ones_v, acc_sh.at[idx_v.at[j]], add=True)
        return 0

    lax.fori_loop(0, J, body, 0)
    plsc.subcore_barrier()
    pltpu.sync_copy(acc_sh.at[pl.ds(s * ROWS_PER_TILE, ROWS_PER_TILE)],
                    out_hbm.at[c, pl.ds(s * ROWS_PER_TILE, ROWS_PER_TILE)])


@functools.lru_cache(maxsize=None)
def _get_deg_kernel():
    mesh = plsc.VectorSubcoreMesh(core_axis_name="c", subcore_axis_name="s",
                                  num_cores=NC, num_subcores=NS)
    return pl.kernel(
        _deg_body,
        out_type=jax.ShapeDtypeStruct((2, R), jnp.float32),
        mesh=mesh,
        scratch_types=[
            pltpu.VMEM((J, K), jnp.int32),
            pltpu.VMEM((K,), jnp.float32),
            pltpu.VMEM((ROWS_PER_TILE,), jnp.float32),
            pltpu.VMEM_SHARED((R,), jnp.float32),
        ],
    )


def _msg_body(src_hbm, dst_hbm, tbl_hbm, out_hbm,
              sidx_v, didx_v, rows0, rows1, acc_sh,
              semr0, semr1, semc0, semc1, sems0, sems1,
              semd0, semd1, semd2, semd3):
    c = lax.axis_index("c")
    s = lax.axis_index("s")

    def fill_z(i, _):
        r = i // (HALF // 16)
        cc = i % (HALF // 16)
        rows0[r, pl.ds(cc * 16, 16)] = jnp.zeros((16,), jnp.float32)
        return 0

    lax.fori_loop(0, K * HALF // 16, fill_z, 0)

    def zero_acc(i, _):
        pltpu.sync_copy(rows0, acc_sh.at[pl.ds(s * ROWS_PER_TILE + i * K, K)])
        return 0

    lax.fori_loop(0, ROWS_PER_TILE // K, zero_acc, 0)
    plsc.subcore_barrier()

    rows = (rows0, rows1)
    semr = (semr0, semr1)
    semc = (semc0, semc1)
    sems = (sems0, sems1)
    semd = (semd0, semd1, semd2, semd3)

    pltpu.sync_copy(src_hbm.at[c, s, 0], sidx_v.at[0])
    pltpu.sync_copy(dst_hbm.at[s, 0], didx_v.at[0])
    pltpu.async_copy(tbl_hbm.at[sidx_v.at[0]], rows0, semr0)
    pltpu.async_copy(src_hbm.at[c, s, 1], sidx_v.at[1], sems1)
    pltpu.async_copy(dst_hbm.at[s, 1], didx_v.at[1], semd1)

    def body(g, _):
        for q in range(4):
            j = 4 * g + q
            br = q % 2
            bn = (q + 1) % 2
            bd = q

            @pl.when(j + 1 < J)
            def _():
                pltpu.make_async_copy(src_hbm.at[c, s, j + 1], sidx_v.at[bn],
                                      sems[bn]).wait()
                pltpu.make_async_copy(dst_hbm.at[s, j + 1],
                                      didx_v.at[(q + 1) % 4],
                                      semd[(q + 1) % 4]).wait()
                if q == 0:
                    @pl.when(j >= 1)
                    def _():
                        pltpu.make_async_copy(
                            rows[bn], acc_sh.at[didx_v.at[(q - 1) % 4]],
                            semc[bn]).wait()
                else:
                    pltpu.make_async_copy(
                        rows[bn], acc_sh.at[didx_v.at[(q - 1) % 4]],
                        semc[bn]).wait()
                pltpu.async_copy(tbl_hbm.at[sidx_v.at[bn]], rows[bn], semr[bn])

            pltpu.make_async_copy(tbl_hbm.at[sidx_v.at[br]], rows[br],
                                  semr[br]).wait()
            pltpu.async_copy(rows[br], acc_sh.at[didx_v.at[bd]], semc[br],
                             add=True)

            @pl.when(j + 2 < J)
            def _():
                pltpu.async_copy(src_hbm.at[c, s, j + 2], sidx_v.at[br],
                                 sems[br])
                pltpu.async_copy(dst_hbm.at[s, j + 2], didx_v.at[(q + 2) % 4],
                                 semd[(q + 2) % 4])
        return 0

    lax.fori_loop(0, J // 4, body, 0)
    pltpu.make_async_copy(rows[0], acc_sh.at[didx_v.at[2]], semc[0]).wait()
    pltpu.make_async_copy(rows[1], acc_sh.at[didx_v.at[3]], semc[1]).wait()
    plsc.subcore_barrier()
    pltpu.sync_copy(acc_sh.at[pl.ds(s * ROWS_PER_TILE, ROWS_PER_TILE)],
                    out_hbm.at[c, pl.ds(s * ROWS_PER_TILE, ROWS_PER_TILE)])


@functools.lru_cache(maxsize=None)
def _get_msg_kernel():
    mesh = plsc.VectorSubcoreMesh(core_axis_name="c", subcore_axis_name="s",
                                  num_cores=NC, num_subcores=NS)
    return pl.kernel(
        _msg_body,
        out_type=jax.ShapeDtypeStruct((2, R, HALF), jnp.float32),
        mesh=mesh,
        scratch_types=[
            pltpu.VMEM((2, K), jnp.int32),
            pltpu.VMEM((4, K), jnp.int32),
            pltpu.VMEM((K, HALF), jnp.float32),
            pltpu.VMEM((K, HALF), jnp.float32),
            pltpu.VMEM_SHARED((R, HALF), jnp.float32),
        ] + [pltpu.SemaphoreType.DMA] * 10,
    )


NB = 1000
GRID = N // NB


def _tca_body(feat_ref, emb_ref, deg_ref, h0_ref, hs_ref, norms_ref):
    nrm = lax.rsqrt(jnp.maximum(deg_ref[0], 1.0))
    norms_ref[0] = nrm
    no = nrm[0]
    h0 = jnp.concatenate(
        [feat_ref[...], jnp.broadcast_to(emb_ref[...], (NB, D_MSG - D_FEAT))],
        axis=1)
    h0_ref[...] = h0
    hs = h0 * no[:, None]
    hs_ref[0] = hs[:, :HALF]
    hs_ref[1] = hs[:, HALF:]


_tca = pl.pallas_call(
    _tca_body,
    grid=(GRID,),
    in_specs=[
        pl.BlockSpec((NB, D_FEAT), lambda i: (i, 0)),
        pl.BlockSpec((1, D_MSG - D_FEAT), lambda i: (0, 0)),
        pl.BlockSpec((1, 2, NB), lambda i: (i, 0, 0)),
    ],
    out_specs=[
        pl.BlockSpec((NB, D_MSG), lambda i: (i, 0)),
        pl.BlockSpec((2, NB, HALF), lambda i: (0, i, 0)),
        pl.BlockSpec((1, 2, NB), lambda i: (i, 0, 0)),
    ],
    out_shape=[
        jax.ShapeDtypeStruct((N, D_MSG), jnp.float32),
        jax.ShapeDtypeStruct((2, N, HALF), jnp.float32),
        jax.ShapeDtypeStruct((GRID, 2, NB), jnp.float32),
    ],
)


def _tcb_body(h_ref, m0_ref, m1_ref, norms_ref, w_ref, b_ref, beta_ref,
              hn_ref, hs_ref=None):
    nin = norms_ref[0, 1]
    mb = jnp.concatenate([m0_ref[0], m1_ref[0]], axis=1) * nin[:, None]
    out = jnp.dot(mb, w_ref[...], preferred_element_type=jnp.float32)
    out = jnp.maximum(out + b_ref[...], 0.0)
    out = h_ref[...] + out
    mu = jnp.mean(out, axis=1, keepdims=True)
    var = jnp.mean((out - mu) ** 2, axis=1, keepdims=True)
    out = (out - mu) * lax.rsqrt(var + EPS) + beta_ref[...]
    hn_ref[...] = out
    if hs_ref is not None:
        no = norms_ref[0, 0]
        hs = out * no[:, None]
        hs_ref[0] = hs[:, :HALF]
        hs_ref[1] = hs[:, HALF:]


def _make_tcb(emit_hs):
    body = _tcb_body if emit_hs else functools.partial(_tcb_body, hs_ref=None)
    out_specs = [pl.BlockSpec((NB, D_MSG), lambda i: (i, 0))]
    out_shape = [jax.ShapeDtypeStruct((N, D_MSG), jnp.float32)]
    if emit_hs:
        out_specs.append(pl.BlockSpec((2, NB, HALF), lambda i: (0, i, 0)))
        out_shape.append(jax.ShapeDtypeStruct((2, N, HALF), jnp.float32))
    return pl.pallas_call(
        body,
        grid=(GRID,),
        in_specs=[
            pl.BlockSpec((NB, D_MSG), lambda i: (i, 0)),
            pl.BlockSpec((1, NB, HALF), lambda i: (0, i, 0)),
            pl.BlockSpec((1, NB, HALF), lambda i: (1, i, 0)),
            pl.BlockSpec((1, 2, NB), lambda i: (i, 0, 0)),
            pl.BlockSpec((D_MSG, D_MSG), lambda i: (0, 0)),
            pl.BlockSpec((1, D_MSG), lambda i: (0, 0)),
            pl.BlockSpec((1, D_MSG), lambda i: (0, 0)),
        ],
        out_specs=out_specs,
        out_shape=out_shape,
    )


_tcb_mid = _make_tcb(True)
_tcb_last = _make_tcb(False)


def kernel(feat, edge_index, emb, W0, b0, beta0, W1, b1, beta1):
    src = edge_index[0].astype(jnp.int32)
    dst = edge_index[1].astype(jnp.int32)
    npad = E_PAD - E

    src0 = jnp.concatenate([src, jnp.zeros((npad,), jnp.int32)])
    msg_src = jnp.stack([src0, src0 + N]).reshape(2, NS, J, K)
    dstp = jnp.concatenate([dst, jnp.full((npad,), TRASH, jnp.int32)])
    msg_dst = dstp.reshape(NS, J, K)
    deg_idx = jnp.stack([
        jnp.concatenate([src, jnp.full((npad,), TRASH, jnp.int32)]),
        dstp,
    ]).reshape(2, NS, J, K)

    deg = _get_deg_kernel()(deg_idx)
    deg3 = deg[:, :N].reshape(2, GRID, NB).transpose(1, 0, 2)
    h0, hs0, norms = _tca(feat, emb, deg3)
    _msg = _get_msg_kernel()
    mA = _msg(msg_src, msg_dst, hs0.reshape(2 * N, HALF))
    h1, hs1 = _tcb_mid(h0, mA, mA, norms, W0, b0.reshape(1, D_MSG),
                       beta0.reshape(1, D_MSG))
    mB = _msg(msg_src, msg_dst, hs1.reshape(2 * N, HALF))
    (h2,) = _tcb_last(h1, mB, mB, norms, W1, b1.reshape(1, D_MSG),
                      beta1.reshape(1, D_MSG))
    return h2
